# baseline (speedup 1.0000x reference)
import functools

import jax
import jax.numpy as jnp
from jax import lax
from jax.experimental import pallas as pl
from jax.experimental.pallas import tpu as pltpu
from jax.experimental.pallas import tpu_sc as plsc

B, N, H, K = 8, 1024, 64, 9
NC, NS = 2, 16
NW = NC * NS
QI = H // 4
JH = H // 2


def _sc_body(idx_hbm, mt_hbm, kt_hbm, vt_hbm, o_hbm,
             ivec, kslab, vslab, buf0, buf1, isem, osem):
    c = lax.axis_index("c")
    s = lax.axis_index("s")
    wid = s * NC + c
    b = wid // 4
    q = wid % 4
    i0 = q * QI
    bufs = (buf0, buf1)

    pltpu.sync_copy(idx_hbm.at[b], ivec)
    pltpu.sync_copy(kt_hbm.at[b, pl.ds(i0, QI)], kslab)

    iv = ivec[...]
    dnums = lax.GatherDimensionNumbers(
        offset_dims=(), collapsed_slice_dims=(0,), start_index_map=(0,)
    )
    idxspl = []
    for kk in range(K):
        sel = jnp.full((16, 1), kk, jnp.int32)
        idxspl.append(
            lax.gather(iv, sel, dnums, (1,),
                       mode=lax.GatherScatterMode.PROMISE_IN_BOUNDS)
        )

    def start_in(cc, p):
        jh, il = divmod(cc, QI)
        pltpu.async_copy(
            mt_hbm.at[b, i0 + il, pl.ds(jh * JH, JH)], bufs[p], isem.at[p]
        )

    def wait_in(cc, p):
        jh, il = divmod(cc, QI)
        pltpu.make_async_copy(
            mt_hbm.at[b, i0 + il, pl.ds(jh * JH, JH)], bufs[p], isem.at[p]
        ).wait()

    def start_out(cc, p):
        jh, il = divmod(cc, QI)
        pltpu.async_copy(
            bufs[p], o_hbm.at[b, i0 + il, pl.ds(jh * JH, JH)], osem.at[p]
        )

    def wait_out(cc, p):
        jh, il = divmod(cc, QI)
        pltpu.make_async_copy(
            bufs[p], o_hbm.at[b, i0 + il, pl.ds(jh * JH, JH)], osem.at[p]
        ).wait()

    jgroups = [jnp.arange(g * 16, (g + 1) * 16, dtype=jnp.int32)
               for g in range(JH // 16)]

    start_in(0, 0)
    start_in(1, 1)
    for cc in range(2 * QI):
        jh, il = divmod(cc, QI)
        p = cc % 2
        if il == 0:
            pltpu.sync_copy(vt_hbm.at[b, pl.ds(jh * JH, JH)], vslab)
        wait_in(cc, p)
        ifull = jnp.full((16,), il, jnp.int32)
        for kk in range(K):
            kvec = plsc.load_gather(kslab, [ifull, idxspl[kk]])
            for g, jv in enumerate(jgroups):
                vvals = plsc.load_gather(vslab, [jv, idxspl[kk]])
                plsc.addupdate_scatter(bufs[p], [jv, idxspl[kk]], kvec * vvals)
        start_out(cc, p)
        if cc + 2 < 2 * QI:
            wait_out(cc, p)
            start_in(cc + 2, p)
    wait_out(2 * QI - 2, 0)
    wait_out(2 * QI - 1, 1)


@jax.jit
def kernel(M, M_k, M_v, indices_update):
    idx = indices_update.astype(jnp.int32)
    idx_pad = jnp.pad(idx, ((0, 0), (0, 16 - K)))
    Mt = jnp.transpose(M, (0, 2, 3, 1))
    Kt = jnp.transpose(M_k, (0, 2, 1))
    Vt = jnp.transpose(M_v, (0, 2, 1))
    f = functools.partial(
        pl.kernel,
        mesh=plsc.VectorSubcoreMesh(core_axis_name="c", subcore_axis_name="s"),
        out_type=jax.ShapeDtypeStruct((B, H, H, N), jnp.float32),
        scratch_types=[
            pltpu.VMEM((16,), jnp.int32),
            pltpu.VMEM((QI, N), jnp.float32),
            pltpu.VMEM((JH, N), jnp.float32),
            pltpu.VMEM((JH, N), jnp.float32),
            pltpu.VMEM((JH, N), jnp.float32),
            pltpu.SemaphoreType.DMA((2,)),
            pltpu.SemaphoreType.DMA((2,)),
        ],
        compiler_params=pltpu.CompilerParams(needs_layout_passes=False),
    )(_sc_body)
    out_t = f(idx_pad, Mt, Kt, Vt)
    return jnp.transpose(out_t, (0, 3, 1, 2))


# speedup vs baseline: 1.0127x; 1.0127x over previous
import functools

import jax
import jax.numpy as jnp
from jax import lax
from jax.experimental import pallas as pl
from jax.experimental.pallas import tpu as pltpu
from jax.experimental.pallas import tpu_sc as plsc

B, N, H, K = 8, 1024, 64, 9
NC, NS = 2, 16
NW = NC * NS
QI = H // 4
JH = 16
NCH = 4 * QI
RING = 3


def _sc_body(idx_hbm, mt_hbm, kt_hbm, vt_hbm, o_hbm,
             ivec, kslab, vslab, buf0, buf1, buf2, isem, osem):
    c = lax.axis_index("c")
    s = lax.axis_index("s")
    wid = s * NC + c
    b = wid // 4
    q = wid % 4
    i0 = q * QI
    bufs = (buf0, buf1, buf2)

    pltpu.sync_copy(idx_hbm.at[b], ivec)
    pltpu.sync_copy(kt_hbm.at[b, pl.ds(i0, QI)], kslab)

    iv = ivec[...]
    dnums = lax.GatherDimensionNumbers(
        offset_dims=(), collapsed_slice_dims=(0,), start_index_map=(0,)
    )
    idxspl = []
    for kk in range(K):
        sel = jnp.full((16, 1), kk, jnp.int32)
        idxspl.append(
            lax.gather(iv, sel, dnums, (1,),
                       mode=lax.GatherScatterMode.PROMISE_IN_BOUNDS)
        )

    def chunk_slice(cc):
        jq, il = divmod(cc, QI)
        return (b, i0 + il, pl.ds(jq * JH, JH))

    def start_in(cc):
        bi, ii, js = chunk_slice(cc)
        pltpu.async_copy(mt_hbm.at[bi, ii, js], bufs[cc % RING],
                         isem.at[cc % RING])

    def wait_in(cc):
        bi, ii, js = chunk_slice(cc)
        pltpu.make_async_copy(mt_hbm.at[bi, ii, js], bufs[cc % RING],
                              isem.at[cc % RING]).wait()

    def start_out(cc):
        bi, ii, js = chunk_slice(cc)
        pltpu.async_copy(bufs[cc % RING], o_hbm.at[bi, ii, js],
                         osem.at[cc % RING])

    def wait_out(cc):
        bi, ii, js = chunk_slice(cc)
        pltpu.make_async_copy(bufs[cc % RING], o_hbm.at[bi, ii, js],
                              osem.at[cc % RING]).wait()

    jv = lax.iota(jnp.int32, 16)

    start_in(0)
    start_in(1)
    for cc in range(NCH):
        jq, il = divmod(cc, QI)
        if il == 0:
            pltpu.sync_copy(vt_hbm.at[b, pl.ds(jq * JH, JH)], vslab)
        wait_in(cc)
        ifull = jnp.full((16,), il, jnp.int32)
        for kk in range(K):
            kvec = plsc.load_gather(kslab, [ifull, idxspl[kk]])
            vvals = plsc.load_gather(vslab, [jv, idxspl[kk]])
            plsc.addupdate_scatter(bufs[cc % RING], [jv, idxspl[kk]],
                                   kvec * vvals)
        start_out(cc)
        if cc + 2 < NCH:
            if cc >= 1:
                wait_out(cc - 1)
            start_in(cc + 2)
    wait_out(NCH - 2)
    wait_out(NCH - 1)


@jax.jit
def kernel(M, M_k, M_v, indices_update):
    idx = indices_update.astype(jnp.int32)
    idx_pad = jnp.pad(idx, ((0, 0), (0, 16 - K)))
    Mt = jnp.transpose(M, (0, 2, 3, 1))
    Kt = jnp.transpose(M_k, (0, 2, 1))
    Vt = jnp.transpose(M_v, (0, 2, 1))
    f = functools.partial(
        pl.kernel,
        mesh=plsc.VectorSubcoreMesh(core_axis_name="c", subcore_axis_name="s"),
        out_type=jax.ShapeDtypeStruct((B, H, H, N), jnp.float32),
        scratch_types=[
            pltpu.VMEM((16,), jnp.int32),
            pltpu.VMEM((QI, N), jnp.float32),
            pltpu.VMEM((JH, N), jnp.float32),
            pltpu.VMEM((JH, N), jnp.float32),
            pltpu.VMEM((JH, N), jnp.float32),
            pltpu.VMEM((JH, N), jnp.float32),
            pltpu.SemaphoreType.DMA((RING,)),
            pltpu.SemaphoreType.DMA((RING,)),
        ],
        compiler_params=pltpu.CompilerParams(needs_layout_passes=False),
    )(_sc_body)
    out_t = f(idx_pad, Mt, Kt, Vt)
    return jnp.transpose(out_t, (0, 3, 1, 2))
